# baseline (speedup 1.0000x reference)
import functools

import jax
import jax.numpy as jnp
from jax import lax
from jax.experimental import pallas as pl
from jax.experimental.pallas import tpu as pltpu
from jax.experimental.pallas import tpu_sc as plsc

_T = 20
_B = 4096
_E = 32
_H = 64
_V = 1000000
_DT = 0.1

_NC, _NS = 2, 16
_NW = _NC * _NS
_TOTAL = _T * _B
_PER_W = _TOTAL // _NW
_CHUNK = 128
_NCHUNK = _PER_W // _CHUNK

_CHL = 16 * 128
_NCHIP = 61
_QL = 976 * 128
_VMAIN = 8 * _QL
_VTAIL = _V - _VMAIN
_VPAD = 640
_VPITCH = _VMAIN + _VPAD

_BLK = 1024


def _detile_body(embt_hbm, tail_hbm, out_hbm, buf_a, buf_b, buf_c, buf_d,
                 tbuf, si0, si1, si2, si3, so0, so1, so2, so3):
    wid = lax.axis_index("s") * _NC + lax.axis_index("c")
    band = jnp.bitwise_and(wid, 3) * 8
    grp = jnp.right_shift(wid, 2)
    base = grp * _QL
    bufs = (buf_a, buf_b, buf_c, buf_d)
    s_in = (si0, si1, si2, si3)
    s_out = (so0, so1, so2, so3)

    @pl.when(grp == 7)
    def _():
        icp = pltpu.make_async_copy(
            tail_hbm.at[pl.ds(band, 8)], tbuf, si0
        )
        icp.start()
        icp.wait()
        for jj in range(8):
            ocp = pltpu.make_async_copy(
                tbuf.at[jj],
                out_hbm.at[pl.ds((band + jj) * _VPITCH + _VMAIN, _VPAD)],
                so0,
            )
            ocp.start()
            ocp.wait()

    plsc.subcore_barrier()

    def in_copy(c, bi):
        return pltpu.make_async_copy(
            embt_hbm.at[pl.ds(band, 8), pl.ds(base + c * _CHL, _CHL)],
            bufs[bi],
            s_in[bi],
        )

    def out_copy(c, bi, jj):
        return pltpu.make_async_copy(
            bufs[bi].at[jj],
            out_hbm.at[pl.ds((band + jj) * _VPITCH + base + c * _CHL, _CHL)],
            s_out[bi],
        )

    in_copy(0, 0).start()
    in_copy(1, 1).start()

    def chunk_step(c, bi):
        in_copy(c, bi).wait()
        for jj in range(8):
            out_copy(c, bi, jj).start()

        @pl.when(c >= 2)
        def _():
            for jj in range(8):
                out_copy(c - 2, (bi + 2) % 4, jj).wait()

        @pl.when(c + 2 < _NCHIP)
        def _():
            in_copy(c + 2, (bi + 2) % 4).start()

    def quad_body(p, _):
        for bi in range(4):
            chunk_step(4 * p + bi, bi)
        return ()

    lax.fori_loop(0, _NCHIP // 4, quad_body, ())
    chunk_step(_NCHIP - 1, (_NCHIP - 1) % 4)
    for jj in range(8):
        out_copy(_NCHIP - 2, (_NCHIP - 2) % 4, jj).wait()
        out_copy(_NCHIP - 1, (_NCHIP - 1) % 4, jj).wait()


def _detile_call(embt, tail_p):
    mesh = plsc.VectorSubcoreMesh(core_axis_name="c", subcore_axis_name="s")
    dk = functools.partial(
        pl.kernel,
        mesh=mesh,
        out_type=jax.ShapeDtypeStruct((_E * _VPITCH,), jnp.float32),
        scratch_types=[
            pltpu.VMEM((8, _CHL), jnp.float32),
            pltpu.VMEM((8, _CHL), jnp.float32),
            pltpu.VMEM((8, _CHL), jnp.float32),
            pltpu.VMEM((8, _CHL), jnp.float32),
            pltpu.VMEM((8, _VPAD), jnp.float32),
            pltpu.SemaphoreType.DMA,
            pltpu.SemaphoreType.DMA,
            pltpu.SemaphoreType.DMA,
            pltpu.SemaphoreType.DMA,
            pltpu.SemaphoreType.DMA,
            pltpu.SemaphoreType.DMA,
            pltpu.SemaphoreType.DMA,
            pltpu.SemaphoreType.DMA,
        ],
    )(_detile_body)
    return dk(embt, tail_p)


def _egather_body(tabt_hbm, idx_hbm, out_hbm, idx_v, col_v, sA, sB, sC, sD):
    wid = lax.axis_index("s") * _NC + lax.axis_index("c")
    pltpu.sync_copy(idx_hbm.at[wid], idx_v)
    sems = (sA, sB, sC, sD)

    def fire(j, s):
        for q in range(_NCHUNK):
            pltpu.async_copy(
                tabt_hbm.at[j].at[idx_v.at[q]],
                col_v.at[j].at[pl.ds(q * _CHUNK, _CHUNK)],
                s,
            )

    def drain(j, s):
        for q in range(_NCHUNK):
            pltpu.make_async_copy(
                tabt_hbm.at[j].at[idx_v.at[q]],
                col_v.at[j].at[pl.ds(q * _CHUNK, _CHUNK)],
                s,
            ).wait()

    for q in range(4):
        fire(q, sems[q])

    def quad_body(p, _):
        for q in range(4):
            j = 4 * p + q
            drain(j, sems[q])

            @pl.when(j + 4 < _E)
            def _():
                fire(j + 4, sems[q])

        return ()

    lax.fori_loop(0, _E // 4, quad_body, ())
    pltpu.sync_copy(col_v, out_hbm.at[pl.ds(0, _E), pl.ds(wid * _PER_W, _PER_W)])


def _egather_call(tabt, idx3):
    mesh = plsc.VectorSubcoreMesh(core_axis_name="c", subcore_axis_name="s")
    gk = functools.partial(
        pl.kernel,
        mesh=mesh,
        out_type=jax.ShapeDtypeStruct((_E, _TOTAL), jnp.float32),
        scratch_types=[
            pltpu.VMEM((_NCHUNK, _CHUNK), jnp.int32),
            pltpu.VMEM((_E, _PER_W), jnp.float32),
            pltpu.SemaphoreType.DMA,
            pltpu.SemaphoreType.DMA,
            pltpu.SemaphoreType.DMA,
            pltpu.SemaphoreType.DMA,
        ],
        compiler_params=pltpu.CompilerParams(use_tc_tiling_on_sc=False),
    )(_egather_body)
    return gk(tabt, idx3)


def _prep_tail(emb):
    tail_t = emb[_VMAIN:, :].T
    return jnp.pad(tail_t, ((0, 0), (0, _VPAD - _VTAIL)))


_NTB = 2
_TPB = _T // _NTB


def _ltc_body(e_ref, win_ref, wrec_ref, b_ref, dec_ref, h_ref, h_scr):
    i = pl.program_id(0)
    win = win_ref[...]
    wrec = wrec_ref[...]
    bb = b_ref[...]
    dec = dec_ref[...]
    dn_t = (((0,), (0,)), ((), ()))

    @pl.when(i == 0)
    def _():
        h_scr[...] = jnp.zeros((_B, _H), jnp.float32)

    h = h_scr[...]
    for s in range(_TPB):
        et = e_ref[:, pl.ds(s * _B, _B)]
        pre = (lax.dot_general(et, win, dn_t,
                               preferred_element_type=jnp.float32)
               + jnp.dot(h, wrec, preferred_element_type=jnp.float32) + bb)
        h = h * dec + _DT * jnp.tanh(pre)
    h_scr[...] = h

    @pl.when(i == _NTB - 1)
    def _():
        h_ref[...] = h


def _ltc_call(eT, w_in, w_rec, b2, dec2):
    return pl.pallas_call(
        _ltc_body,
        grid=(_NTB,),
        in_specs=[
            pl.BlockSpec((_E, _TPB * _B), lambda i: (0, i)),
            pl.BlockSpec((_E, _H), lambda i: (0, 0)),
            pl.BlockSpec((_H, _H), lambda i: (0, 0)),
            pl.BlockSpec((1, _H), lambda i: (0, 0)),
            pl.BlockSpec((1, _H), lambda i: (0, 0)),
        ],
        out_specs=pl.BlockSpec((_B, _H), lambda i: (0, 0)),
        out_shape=jax.ShapeDtypeStruct((_B, _H), jnp.float32),
        scratch_shapes=[pltpu.VMEM((_B, _H), jnp.float32)],
    )(eT, w_in, w_rec, b2, dec2)


def kernel(x, emb, W_in, W_rec, b, tau):
    cm = _detile_call(emb.T, _prep_tail(emb))
    tabt = cm.reshape(_E, _VPITCH)
    idx3 = x.reshape(-1).astype(jnp.int32).reshape(_NW, _NCHUNK, _CHUNK)
    eT = _egather_call(tabt, idx3)
    dec2 = (1.0 - _DT / tau).reshape(1, _H)
    return _ltc_call(eT, W_in, W_rec, b.reshape(1, _H), dec2)


# speedup vs baseline: 1.0155x; 1.0155x over previous
import functools

import jax
import jax.numpy as jnp
from jax import lax
from jax.experimental import pallas as pl
from jax.experimental.pallas import tpu as pltpu
from jax.experimental.pallas import tpu_sc as plsc

_T = 20
_B = 4096
_E = 32
_H = 64
_V = 1000000
_DT = 0.1

_NC, _NS = 2, 16
_NW = _NC * _NS
_TOTAL = _T * _B
_PER_W = _TOTAL // _NW
_CHUNK = 128
_NCHUNK = _PER_W // _CHUNK

_CHL = 61 * 128
_NCHIP = 16
_QL = 976 * 128
_VMAIN = 8 * _QL
_VTAIL = _V - _VMAIN
_VPAD = 640
_VPITCH = _VMAIN + _VPAD

_BLK = 1024


def _detile_body(embt_hbm, tail_hbm, out_hbm, buf_a, buf_b, tbuf,
                 si0, si1, so0, so1):
    wid = lax.axis_index("s") * _NC + lax.axis_index("c")
    band = jnp.bitwise_and(wid, 3) * 8
    grp = jnp.right_shift(wid, 2)
    base = grp * _QL
    bufs = (buf_a, buf_b)
    s_in = (si0, si1)
    s_out = (so0, so1)

    @pl.when(grp == 7)
    def _():
        icp = pltpu.make_async_copy(
            tail_hbm.at[pl.ds(band, 8)], tbuf, si0
        )
        icp.start()
        icp.wait()
        for jj in range(8):
            ocp = pltpu.make_async_copy(
                tbuf.at[jj],
                out_hbm.at[pl.ds((band + jj) * _VPITCH + _VMAIN, _VPAD)],
                so0,
            )
            ocp.start()
            ocp.wait()

    plsc.subcore_barrier()

    def in_copy(c, bi):
        return pltpu.make_async_copy(
            embt_hbm.at[pl.ds(band, 8), pl.ds(base + c * _CHL, _CHL)],
            bufs[bi],
            s_in[bi],
        )

    def out_copy(c, bi, jj):
        return pltpu.make_async_copy(
            bufs[bi].at[jj],
            out_hbm.at[pl.ds((band + jj) * _VPITCH + base + c * _CHL, _CHL)],
            s_out[bi],
        )

    in_copy(0, 0).start()

    def chunk_step(c, bi):
        ob = 1 - bi
        in_copy(c, bi).wait()

        @pl.when(c >= 1)
        def _():
            for jj in range(8):
                out_copy(c - 1, ob, jj).wait()

        @pl.when(c + 1 < _NCHIP)
        def _():
            in_copy(c + 1, ob).start()

        for jj in range(8):
            out_copy(c, bi, jj).start()

    def pair_body(p, _):
        chunk_step(2 * p, 0)
        chunk_step(2 * p + 1, 1)
        return ()

    lax.fori_loop(0, _NCHIP // 2, pair_body, ())
    for jj in range(8):
        out_copy(_NCHIP - 1, 1, jj).wait()


def _detile_call(embt, tail_p):
    mesh = plsc.VectorSubcoreMesh(core_axis_name="c", subcore_axis_name="s")
    dk = functools.partial(
        pl.kernel,
        mesh=mesh,
        out_type=jax.ShapeDtypeStruct((_E * _VPITCH,), jnp.float32),
        scratch_types=[
            pltpu.VMEM((8, _CHL), jnp.float32),
            pltpu.VMEM((8, _CHL), jnp.float32),
            pltpu.VMEM((8, _VPAD), jnp.float32),
            pltpu.SemaphoreType.DMA,
            pltpu.SemaphoreType.DMA,
            pltpu.SemaphoreType.DMA,
            pltpu.SemaphoreType.DMA,
        ],
    )(_detile_body)
    return dk(embt, tail_p)


def _egather_body(tabt_hbm, idx_hbm, out_hbm, idx_v, col_v, sA, sB, sC, sD):
    wid = lax.axis_index("s") * _NC + lax.axis_index("c")
    pltpu.sync_copy(idx_hbm.at[wid], idx_v)
    sems = (sA, sB, sC, sD)

    def fire(j, s):
        for q in range(_NCHUNK):
            pltpu.async_copy(
                tabt_hbm.at[j].at[idx_v.at[q]],
                col_v.at[j].at[pl.ds(q * _CHUNK, _CHUNK)],
                s,
            )

    def drain(j, s):
        for q in range(_NCHUNK):
            pltpu.make_async_copy(
                tabt_hbm.at[j].at[idx_v.at[q]],
                col_v.at[j].at[pl.ds(q * _CHUNK, _CHUNK)],
                s,
            ).wait()

    for q in range(4):
        fire(q, sems[q])

    def quad_body(p, _):
        for q in range(4):
            j = 4 * p + q
            drain(j, sems[q])

            @pl.when(j + 4 < _E)
            def _():
                fire(j + 4, sems[q])

        return ()

    lax.fori_loop(0, _E // 4, quad_body, ())
    pltpu.sync_copy(col_v, out_hbm.at[pl.ds(0, _E), pl.ds(wid * _PER_W, _PER_W)])


def _egather_call(tabt, idx3):
    mesh = plsc.VectorSubcoreMesh(core_axis_name="c", subcore_axis_name="s")
    gk = functools.partial(
        pl.kernel,
        mesh=mesh,
        out_type=jax.ShapeDtypeStruct((_E, _TOTAL), jnp.float32),
        scratch_types=[
            pltpu.VMEM((_NCHUNK, _CHUNK), jnp.int32),
            pltpu.VMEM((_E, _PER_W), jnp.float32),
            pltpu.SemaphoreType.DMA,
            pltpu.SemaphoreType.DMA,
            pltpu.SemaphoreType.DMA,
            pltpu.SemaphoreType.DMA,
        ],
        compiler_params=pltpu.CompilerParams(use_tc_tiling_on_sc=False),
    )(_egather_body)
    return gk(tabt, idx3)


def _prep_tail(emb):
    tail_t = emb[_VMAIN:, :].T
    return jnp.pad(tail_t, ((0, 0), (0, _VPAD - _VTAIL)))


_NTB = 4
_TPB = _T // _NTB


def _ltc_body(e_ref, win_ref, wrec_ref, b_ref, dec_ref, h_ref, h_scr):
    i = pl.program_id(0)
    win = win_ref[...]
    wrec = wrec_ref[...]
    bb = b_ref[...]
    dec = dec_ref[...]
    dn_t = (((0,), (0,)), ((), ()))

    @pl.when(i == 0)
    def _():
        h_scr[...] = jnp.zeros((_B, _H), jnp.float32)

    h = h_scr[...]
    for s in range(_TPB):
        et = e_ref[:, pl.ds(s * _B, _B)]
        pre = (lax.dot_general(et, win, dn_t,
                               preferred_element_type=jnp.float32)
               + jnp.dot(h, wrec, preferred_element_type=jnp.float32) + bb)
        h = h * dec + _DT * jnp.tanh(pre)
    h_scr[...] = h

    @pl.when(i == _NTB - 1)
    def _():
        h_ref[...] = h


def _ltc_call(eT, w_in, w_rec, b2, dec2):
    return pl.pallas_call(
        _ltc_body,
        grid=(_NTB,),
        in_specs=[
            pl.BlockSpec((_E, _TPB * _B), lambda i: (0, i)),
            pl.BlockSpec((_E, _H), lambda i: (0, 0)),
            pl.BlockSpec((_H, _H), lambda i: (0, 0)),
            pl.BlockSpec((1, _H), lambda i: (0, 0)),
            pl.BlockSpec((1, _H), lambda i: (0, 0)),
        ],
        out_specs=pl.BlockSpec((_B, _H), lambda i: (0, 0)),
        out_shape=jax.ShapeDtypeStruct((_B, _H), jnp.float32),
        scratch_shapes=[pltpu.VMEM((_B, _H), jnp.float32)],
    )(eT, w_in, w_rec, b2, dec2)


def kernel(x, emb, W_in, W_rec, b, tau):
    cm = _detile_call(emb.T, _prep_tail(emb))
    tabt = cm.reshape(_E, _VPITCH)
    idx3 = x.reshape(-1).astype(jnp.int32).reshape(_NW, _NCHUNK, _CHUNK)
    eT = _egather_call(tabt, idx3)
    dec2 = (1.0 - _DT / tau).reshape(1, _H)
    return _ltc_call(eT, W_in, W_rec, b.reshape(1, _H), dec2)


# speedup vs baseline: 1.0186x; 1.0031x over previous
import functools

import jax
import jax.numpy as jnp
from jax import lax
from jax.experimental import pallas as pl
from jax.experimental.pallas import tpu as pltpu
from jax.experimental.pallas import tpu_sc as plsc

_T = 20
_B = 4096
_E = 32
_H = 64
_V = 1000000
_DT = 0.1

_NC, _NS = 2, 16
_NW = _NC * _NS
_TOTAL = _T * _B
_PER_W = _TOTAL // _NW
_CHUNK = 128
_NCHUNK = _PER_W // _CHUNK

_CHL = 61 * 128
_NCHIP = 16
_QL = 976 * 128
_VMAIN = 8 * _QL
_VTAIL = _V - _VMAIN
_VPAD = 640
_VPITCH = _VMAIN + _VPAD

_BLK = 1024


def _detile_body(embt_hbm, tail_hbm, out_hbm, buf_a, buf_b, tbuf,
                 si0, si1, so0, so1):
    wid = lax.axis_index("s") * _NC + lax.axis_index("c")
    band = jnp.bitwise_and(wid, 3) * 8
    grp = jnp.right_shift(wid, 2)
    base = grp * _QL
    bufs = (buf_a, buf_b)
    s_in = (si0, si1)
    s_out = (so0, so1)

    @pl.when(grp == 7)
    def _():
        icp = pltpu.make_async_copy(
            tail_hbm.at[pl.ds(band, 8)], tbuf, si0
        )
        icp.start()
        icp.wait()
        for jj in range(8):
            ocp = pltpu.make_async_copy(
                tbuf.at[jj],
                out_hbm.at[pl.ds((band + jj) * _VPITCH + _VMAIN, _VPAD)],
                so0,
            )
            ocp.start()
            ocp.wait()

    plsc.subcore_barrier()

    def in_copy(c, bi):
        return pltpu.make_async_copy(
            embt_hbm.at[pl.ds(band, 8), pl.ds(base + c * _CHL, _CHL)],
            bufs[bi],
            s_in[bi],
        )

    def out_copy(c, bi, jj):
        return pltpu.make_async_copy(
            bufs[bi].at[jj],
            out_hbm.at[pl.ds((band + jj) * _VPITCH + base + c * _CHL, _CHL)],
            s_out[bi],
        )

    in_copy(0, 0).start()

    def chunk_step(c, bi):
        ob = 1 - bi
        in_copy(c, bi).wait()

        @pl.when(c >= 1)
        def _():
            for jj in range(8):
                out_copy(c - 1, ob, jj).wait()

        @pl.when(c + 1 < _NCHIP)
        def _():
            in_copy(c + 1, ob).start()

        for jj in range(8):
            out_copy(c, bi, jj).start()

    def pair_body(p, _):
        chunk_step(2 * p, 0)
        chunk_step(2 * p + 1, 1)
        return ()

    lax.fori_loop(0, _NCHIP // 2, pair_body, ())
    for jj in range(8):
        out_copy(_NCHIP - 1, 1, jj).wait()


def _detile_call(embt, tail_p):
    mesh = plsc.VectorSubcoreMesh(core_axis_name="c", subcore_axis_name="s")
    dk = functools.partial(
        pl.kernel,
        mesh=mesh,
        out_type=jax.ShapeDtypeStruct((_E * _VPITCH,), jnp.float32),
        scratch_types=[
            pltpu.VMEM((8, _CHL), jnp.float32),
            pltpu.VMEM((8, _CHL), jnp.float32),
            pltpu.VMEM((8, _VPAD), jnp.float32),
            pltpu.SemaphoreType.DMA,
            pltpu.SemaphoreType.DMA,
            pltpu.SemaphoreType.DMA,
            pltpu.SemaphoreType.DMA,
        ],
    )(_detile_body)
    return dk(embt, tail_p)


def _egather_body(tabt_hbm, idx_hbm, out_hbm, idx_v, col_v, sA, sB, sC, sD):
    wid = lax.axis_index("s") * _NC + lax.axis_index("c")
    pltpu.sync_copy(idx_hbm.at[wid], idx_v)
    sems = (sA, sB, sC, sD)

    def fire(j, s):
        pltpu.async_copy(tabt_hbm.at[j].at[idx_v], col_v.at[j], s)

    def drain(j, s):
        pltpu.make_async_copy(
            tabt_hbm.at[j].at[idx_v], col_v.at[j], s
        ).wait()

    for q in range(4):
        fire(q, sems[q])

    def quad_body(p, _):
        for q in range(4):
            j = 4 * p + q
            drain(j, sems[q])

            @pl.when(j + 4 < _E)
            def _():
                fire(j + 4, sems[q])

        return ()

    lax.fori_loop(0, _E // 4, quad_body, ())
    pltpu.sync_copy(col_v, out_hbm.at[pl.ds(0, _E), pl.ds(wid * _PER_W, _PER_W)])


def _egather_call(tabt, idx2):
    mesh = plsc.VectorSubcoreMesh(core_axis_name="c", subcore_axis_name="s")
    gk = functools.partial(
        pl.kernel,
        mesh=mesh,
        out_type=jax.ShapeDtypeStruct((_E, _TOTAL), jnp.float32),
        scratch_types=[
            pltpu.VMEM((_PER_W,), jnp.int32),
            pltpu.VMEM((_E, _PER_W), jnp.float32),
            pltpu.SemaphoreType.DMA,
            pltpu.SemaphoreType.DMA,
            pltpu.SemaphoreType.DMA,
            pltpu.SemaphoreType.DMA,
        ],
        compiler_params=pltpu.CompilerParams(use_tc_tiling_on_sc=False),
    )(_egather_body)
    return gk(tabt, idx2)


def _prep_tail(emb):
    tail_t = emb[_VMAIN:, :].T
    return jnp.pad(tail_t, ((0, 0), (0, _VPAD - _VTAIL)))


_NTB = 4
_TPB = _T // _NTB


def _ltc_body(e_ref, win_ref, wrec_ref, b_ref, dec_ref, h_ref, h_scr):
    i = pl.program_id(0)
    win = win_ref[...]
    wrec = wrec_ref[...]
    bb = b_ref[...]
    dec = dec_ref[...]
    dn_t = (((0,), (0,)), ((), ()))

    @pl.when(i == 0)
    def _():
        h_scr[...] = jnp.zeros((_B, _H), jnp.float32)

    h = h_scr[...]
    for s in range(_TPB):
        et = e_ref[:, pl.ds(s * _B, _B)]
        pre = (lax.dot_general(et, win, dn_t,
                               preferred_element_type=jnp.float32)
               + jnp.dot(h, wrec, preferred_element_type=jnp.float32) + bb)
        h = h * dec + _DT * jnp.tanh(pre)
    h_scr[...] = h

    @pl.when(i == _NTB - 1)
    def _():
        h_ref[...] = h


def _ltc_call(eT, w_in, w_rec, b2, dec2):
    return pl.pallas_call(
        _ltc_body,
        grid=(_NTB,),
        in_specs=[
            pl.BlockSpec((_E, _TPB * _B), lambda i: (0, i)),
            pl.BlockSpec((_E, _H), lambda i: (0, 0)),
            pl.BlockSpec((_H, _H), lambda i: (0, 0)),
            pl.BlockSpec((1, _H), lambda i: (0, 0)),
            pl.BlockSpec((1, _H), lambda i: (0, 0)),
        ],
        out_specs=pl.BlockSpec((_B, _H), lambda i: (0, 0)),
        out_shape=jax.ShapeDtypeStruct((_B, _H), jnp.float32),
        scratch_shapes=[pltpu.VMEM((_B, _H), jnp.float32)],
    )(eT, w_in, w_rec, b2, dec2)


def kernel(x, emb, W_in, W_rec, b, tau):
    cm = _detile_call(emb.T, _prep_tail(emb))
    tabt = cm.reshape(_E, _VPITCH)
    idx2 = x.reshape(-1).astype(jnp.int32).reshape(_NW, _PER_W)
    eT = _egather_call(tabt, idx2)
    dec2 = (1.0 - _DT / tau).reshape(1, _H)
    return _ltc_call(eT, W_in, W_rec, b.reshape(1, _H), dec2)
